# baseline (speedup 1.0000x reference)
import functools

import jax
import jax.numpy as jnp
from jax import lax
from jax.experimental import pallas as pl
from jax.experimental.pallas import tpu as pltpu
from jax.experimental.pallas import tpu_sc as plsc

EMBED_COUNT = 100000
ES = 64
NF = 128
OUT_W = NF + ES
TW = 128

NC, NS = 2, 16
NW = NC * NS
CHUNK = 128
K = 4

BM = 256


def _make_gather_kernel(N: int):
    rows_per_w = N // NW
    steps = rows_per_w // CHUNK
    assert steps % K == 0
    mesh = plsc.VectorSubcoreMesh(
        core_axis_name="c", subcore_axis_name="s", num_cores=NC, num_subcores=NS
    )

    @functools.partial(
        pl.kernel,
        out_type=jax.ShapeDtypeStruct((N, TW), jnp.float32),
        mesh=mesh,
        scratch_types=[
            pltpu.VMEM((rows_per_w,), jnp.int32),
            [pltpu.VMEM((CHUNK, TW), jnp.float32)] * K,
            [pltpu.SemaphoreType.DMA] * K,
            [pltpu.SemaphoreType.DMA] * K,
        ],
        compiler_params=pltpu.CompilerParams(use_tc_tiling_on_sc=True),
    )
    def gather_kernel(idx_hbm, table_hbm, out_hbm, idx_v, gbufs, gsems, wsems):
        wid = lax.axis_index("s") * NC + lax.axis_index("c")
        base = wid * rows_per_w
        pltpu.sync_copy(idx_hbm.at[pl.ds(base, rows_per_w)], idx_v)

        def group(g, _):
            s0 = g * K
            for b in range(K):
                pltpu.async_copy(
                    table_hbm.at[idx_v.at[pl.ds((s0 + b) * CHUNK, CHUNK)]],
                    gbufs[b], gsems[b],
                )
            for b in range(K):
                r0 = base + (s0 + b) * CHUNK
                pltpu.make_async_copy(
                    table_hbm.at[idx_v.at[pl.ds((s0 + b) * CHUNK, CHUNK)]],
                    gbufs[b], gsems[b],
                ).wait()
                pltpu.async_copy(
                    gbufs[b], out_hbm.at[pl.ds(r0, CHUNK)], wsems[b],
                )
            for b in range(K):
                r0 = base + (s0 + b) * CHUNK
                pltpu.make_async_copy(
                    gbufs[b], out_hbm.at[pl.ds(r0, CHUNK)], wsems[b],
                ).wait()
            return ()

        lax.fori_loop(0, steps // K, group, (), unroll=False)

    return gather_kernel


LB = 8


def _dense_body(wd_ref, out_ref):
    for l in range(LB):
        out_ref[l, :, :] = wd_ref[:, l, :].T


def _dense_t(wd3):
    B, L, _ = wd3.shape
    return pl.pallas_call(
        _dense_body,
        grid=(L // LB, B // BM),
        in_specs=[
            pl.BlockSpec((BM, LB, NF), lambda i, j: (j, i, 0)),
        ],
        out_specs=pl.BlockSpec((LB, NF, BM), lambda i, j: (i, 0, j)),
        out_shape=jax.ShapeDtypeStruct((L, OUT_W, B), jnp.float32),
        compiler_params=pltpu.CompilerParams(
            dimension_semantics=("arbitrary", "arbitrary"),
        ),
    )(wd3)


def _embed_body(emb_ref, prev_ref, out_ref):
    del prev_ref
    for l in range(LB):
        out_ref[l, :, :] = emb_ref[:, l, :ES].T


def _embed_t(embj3, out_t):
    B, L, _ = embj3.shape
    return pl.pallas_call(
        _embed_body,
        grid=(L // LB, B // BM),
        in_specs=[
            pl.BlockSpec((BM, LB, TW), lambda i, j: (j, i, 0)),
            pl.BlockSpec(memory_space=pl.ANY),
        ],
        out_specs=pl.BlockSpec((LB, ES, BM), lambda i, j: (i, 2, j)),
        out_shape=jax.ShapeDtypeStruct((L, OUT_W, B), jnp.float32),
        input_output_aliases={1: 0},
        compiler_params=pltpu.CompilerParams(
            dimension_semantics=("arbitrary", "arbitrary"),
        ),
    )(embj3, out_t)


def kernel(wall_distances, keymask, key_embed):
    B, L, F = wall_distances.shape
    N = B * L
    km = jnp.squeeze(keymask, axis=2).astype(jnp.int32)
    km = jnp.where(km > EMBED_COUNT, 0, km)
    idx = km.reshape(N)
    table = jnp.pad(key_embed, ((0, 0), (0, TW - ES)))
    embj3 = _make_gather_kernel(N)(idx, table).reshape(B, L, TW)
    out_t = _dense_t(wall_distances)
    out_t = _embed_t(embj3, out_t)
    return jnp.transpose(out_t, (2, 0, 1))


# speedup vs baseline: 1.0924x; 1.0924x over previous
import functools

import jax
import jax.numpy as jnp
from jax import lax
from jax.experimental import pallas as pl
from jax.experimental.pallas import tpu as pltpu
from jax.experimental.pallas import tpu_sc as plsc

EMBED_COUNT = 100000
ES = 64
NF = 128
OUT_W = NF + ES
TW = 128

NC, NS = 2, 16
NW = NC * NS
CHUNK = 128
K = 4

BM = 256


def _make_gather_kernel(N: int):
    rows_per_w = N // NW
    steps = rows_per_w // CHUNK
    assert steps % K == 0
    mesh = plsc.VectorSubcoreMesh(
        core_axis_name="c", subcore_axis_name="s", num_cores=NC, num_subcores=NS
    )

    @functools.partial(
        pl.kernel,
        out_type=jax.ShapeDtypeStruct((N, TW), jnp.float32),
        mesh=mesh,
        scratch_types=[
            pltpu.VMEM((rows_per_w,), jnp.int32),
            [pltpu.VMEM((CHUNK, TW), jnp.float32)] * K,
            [pltpu.SemaphoreType.DMA] * K,
            [pltpu.SemaphoreType.DMA] * K,
        ],
        compiler_params=pltpu.CompilerParams(use_tc_tiling_on_sc=True),
    )
    def gather_kernel(idx_hbm, table_hbm, out_hbm, idx_v, gbufs, gsems, wsems):
        wid = lax.axis_index("s") * NC + lax.axis_index("c")
        base = wid * rows_per_w
        pltpu.sync_copy(idx_hbm.at[pl.ds(base, rows_per_w)], idx_v)

        def group(g, _):
            s0 = g * K
            for b in range(K):
                pltpu.async_copy(
                    table_hbm.at[idx_v.at[pl.ds((s0 + b) * CHUNK, CHUNK)]],
                    gbufs[b], gsems[b],
                )
            for b in range(K):
                r0 = base + (s0 + b) * CHUNK
                pltpu.make_async_copy(
                    table_hbm.at[idx_v.at[pl.ds((s0 + b) * CHUNK, CHUNK)]],
                    gbufs[b], gsems[b],
                ).wait()
                pltpu.async_copy(
                    gbufs[b], out_hbm.at[pl.ds(r0, CHUNK)], wsems[b],
                )
            for b in range(K):
                r0 = base + (s0 + b) * CHUNK
                pltpu.make_async_copy(
                    gbufs[b], out_hbm.at[pl.ds(r0, CHUNK)], wsems[b],
                ).wait()
            return ()

        lax.fori_loop(0, steps // K, group, (), unroll=False)

    return gather_kernel


LB = 8


def _concat_body(wd_ref, emb_ref, out_ref):
    for l in range(LB):
        out_ref[l, :NF, :] = wd_ref[:, l, :].T
        out_ref[l, NF:, :] = emb_ref[:, l, :ES].T


def _concat_t(wd3, embj3):
    B, L, _ = wd3.shape
    return pl.pallas_call(
        _concat_body,
        grid=(L // LB, B // BM),
        in_specs=[
            pl.BlockSpec((BM, LB, NF), lambda i, j: (j, i, 0)),
            pl.BlockSpec((BM, LB, TW), lambda i, j: (j, i, 0)),
        ],
        out_specs=pl.BlockSpec((LB, OUT_W, BM), lambda i, j: (i, 0, j)),
        out_shape=jax.ShapeDtypeStruct((L, OUT_W, B), jnp.float32),
        compiler_params=pltpu.CompilerParams(
            dimension_semantics=("arbitrary", "arbitrary"),
        ),
    )(wd3, embj3)


def kernel(wall_distances, keymask, key_embed):
    B, L, F = wall_distances.shape
    N = B * L
    km = jnp.squeeze(keymask, axis=2).astype(jnp.int32)
    km = jnp.where(km > EMBED_COUNT, 0, km)
    idx = km.reshape(N)
    table = jnp.pad(key_embed, ((0, 0), (0, TW - ES)))
    embj3 = _make_gather_kernel(N)(idx, table).reshape(B, L, TW)
    out_t = _concat_t(wall_distances, embj3)
    return jnp.transpose(out_t, (2, 0, 1))


# speedup vs baseline: 1.2259x; 1.1222x over previous
import functools

import jax
import jax.numpy as jnp
from jax import lax
from jax.experimental import pallas as pl
from jax.experimental.pallas import tpu as pltpu
from jax.experimental.pallas import tpu_sc as plsc

EMBED_COUNT = 100000
ES = 64
NF = 128
OUT_W = NF + ES
TW = 128

NC, NS = 2, 16
NW = NC * NS
CHUNK = 128
K = 4

BM = 512


def _make_gather_kernel(N: int):
    rows_per_w = N // NW
    steps = rows_per_w // CHUNK
    assert steps % K == 0
    mesh = plsc.VectorSubcoreMesh(
        core_axis_name="c", subcore_axis_name="s", num_cores=NC, num_subcores=NS
    )

    @functools.partial(
        pl.kernel,
        out_type=jax.ShapeDtypeStruct((N, TW), jnp.float32),
        mesh=mesh,
        scratch_types=[
            pltpu.VMEM((rows_per_w,), jnp.int32),
            [pltpu.VMEM((CHUNK, TW), jnp.float32)] * K,
            [pltpu.SemaphoreType.DMA] * K,
            [pltpu.SemaphoreType.DMA] * K,
        ],
        compiler_params=pltpu.CompilerParams(use_tc_tiling_on_sc=True),
    )
    def gather_kernel(idx_hbm, table_hbm, out_hbm, idx_v, gbufs, gsems, wsems):
        wid = lax.axis_index("s") * NC + lax.axis_index("c")
        base = wid * rows_per_w
        pltpu.sync_copy(idx_hbm.at[pl.ds(base, rows_per_w)], idx_v)

        def group(g, _):
            s0 = g * K
            for b in range(K):
                pltpu.async_copy(
                    table_hbm.at[idx_v.at[pl.ds((s0 + b) * CHUNK, CHUNK)]],
                    gbufs[b], gsems[b],
                )
            for b in range(K):
                r0 = base + (s0 + b) * CHUNK
                pltpu.make_async_copy(
                    table_hbm.at[idx_v.at[pl.ds((s0 + b) * CHUNK, CHUNK)]],
                    gbufs[b], gsems[b],
                ).wait()
                pltpu.async_copy(
                    gbufs[b], out_hbm.at[pl.ds(r0, CHUNK)], wsems[b],
                )
            for b in range(K):
                r0 = base + (s0 + b) * CHUNK
                pltpu.make_async_copy(
                    gbufs[b], out_hbm.at[pl.ds(r0, CHUNK)], wsems[b],
                ).wait()
            return ()

        lax.fori_loop(0, steps // K, group, (), unroll=False)

    return gather_kernel


LB = 8


def _concat_body(wd_ref, emb_ref, out_ref):
    for l in range(LB):
        out_ref[l, :NF, :] = wd_ref[:, l, :].T
        out_ref[l, NF:, :] = emb_ref[:, l, :ES].T


def _concat_t(wd3, embj3):
    B, L, _ = wd3.shape
    return pl.pallas_call(
        _concat_body,
        grid=(L // LB, B // BM),
        in_specs=[
            pl.BlockSpec((BM, LB, NF), lambda i, j: (j, i, 0)),
            pl.BlockSpec((BM, LB, TW), lambda i, j: (j, i, 0)),
        ],
        out_specs=pl.BlockSpec((LB, OUT_W, BM), lambda i, j: (i, 0, j)),
        out_shape=jax.ShapeDtypeStruct((L, OUT_W, B), jnp.float32),
        compiler_params=pltpu.CompilerParams(
            dimension_semantics=("arbitrary", "arbitrary"),
        ),
    )(wd3, embj3)


def kernel(wall_distances, keymask, key_embed):
    B, L, F = wall_distances.shape
    N = B * L
    km = jnp.squeeze(keymask, axis=2).astype(jnp.int32)
    km = jnp.where(km > EMBED_COUNT, 0, km)
    idx = km.reshape(N)
    table = jnp.pad(key_embed, ((0, 0), (0, TW - ES)))
    embj3 = _make_gather_kernel(N)(idx, table).reshape(B, L, TW)
    out_t = _concat_t(wall_distances, embj3)
    return jnp.transpose(out_t, (2, 0, 1))


# speedup vs baseline: 1.3111x; 1.0695x over previous
import functools

import jax
import jax.numpy as jnp
from jax import lax
from jax.experimental import pallas as pl
from jax.experimental.pallas import tpu as pltpu
from jax.experimental.pallas import tpu_sc as plsc

EMBED_COUNT = 100000
ES = 64
NF = 128
OUT_W = NF + ES
TW = 128

NC, NS = 2, 16
NW = NC * NS
CHUNK = 128
K = 4

BM = 1024


def _make_gather_kernel(N: int):
    rows_per_w = N // NW
    steps = rows_per_w // CHUNK
    assert steps % K == 0
    mesh = plsc.VectorSubcoreMesh(
        core_axis_name="c", subcore_axis_name="s", num_cores=NC, num_subcores=NS
    )

    @functools.partial(
        pl.kernel,
        out_type=jax.ShapeDtypeStruct((N, TW), jnp.float32),
        mesh=mesh,
        scratch_types=[
            pltpu.VMEM((rows_per_w,), jnp.int32),
            [pltpu.VMEM((CHUNK, TW), jnp.float32)] * K,
            [pltpu.SemaphoreType.DMA] * K,
            [pltpu.SemaphoreType.DMA] * K,
        ],
        compiler_params=pltpu.CompilerParams(use_tc_tiling_on_sc=True),
    )
    def gather_kernel(idx_hbm, table_hbm, out_hbm, idx_v, gbufs, gsems, wsems):
        wid = lax.axis_index("s") * NC + lax.axis_index("c")
        base = wid * rows_per_w
        pltpu.sync_copy(idx_hbm.at[pl.ds(base, rows_per_w)], idx_v)

        def group(g, _):
            s0 = g * K
            for b in range(K):
                pltpu.async_copy(
                    table_hbm.at[idx_v.at[pl.ds((s0 + b) * CHUNK, CHUNK)]],
                    gbufs[b], gsems[b],
                )
            for b in range(K):
                r0 = base + (s0 + b) * CHUNK
                pltpu.make_async_copy(
                    table_hbm.at[idx_v.at[pl.ds((s0 + b) * CHUNK, CHUNK)]],
                    gbufs[b], gsems[b],
                ).wait()
                pltpu.async_copy(
                    gbufs[b], out_hbm.at[pl.ds(r0, CHUNK)], wsems[b],
                )
            for b in range(K):
                r0 = base + (s0 + b) * CHUNK
                pltpu.make_async_copy(
                    gbufs[b], out_hbm.at[pl.ds(r0, CHUNK)], wsems[b],
                ).wait()
            return ()

        lax.fori_loop(0, steps // K, group, (), unroll=False)

    return gather_kernel


LB = 8


def _concat_body(wd_ref, emb_ref, out_ref):
    for l in range(LB):
        out_ref[l, :NF, :] = wd_ref[:, l, :].T
        out_ref[l, NF:, :] = emb_ref[:, l, :ES].T


def _concat_t(wd3, embj3):
    B, L, _ = wd3.shape
    return pl.pallas_call(
        _concat_body,
        grid=(L // LB, B // BM),
        in_specs=[
            pl.BlockSpec((BM, LB, NF), lambda i, j: (j, i, 0)),
            pl.BlockSpec((BM, LB, TW), lambda i, j: (j, i, 0)),
        ],
        out_specs=pl.BlockSpec((LB, OUT_W, BM), lambda i, j: (i, 0, j)),
        out_shape=jax.ShapeDtypeStruct((L, OUT_W, B), jnp.float32),
        compiler_params=pltpu.CompilerParams(
            dimension_semantics=("arbitrary", "arbitrary"),
        ),
    )(wd3, embj3)


def kernel(wall_distances, keymask, key_embed):
    B, L, F = wall_distances.shape
    N = B * L
    km = jnp.squeeze(keymask, axis=2).astype(jnp.int32)
    km = jnp.where(km > EMBED_COUNT, 0, km)
    idx = km.reshape(N)
    table = jnp.pad(key_embed, ((0, 0), (0, TW - ES)))
    embj3 = _make_gather_kernel(N)(idx, table).reshape(B, L, TW)
    out_t = _concat_t(wall_distances, embj3)
    return jnp.transpose(out_t, (2, 0, 1))
